# baseline (speedup 1.0000x reference)
import functools

import jax
import jax.numpy as jnp
from jax import lax
from jax.experimental import pallas as pl
from jax.experimental.pallas import tpu as pltpu
from jax.experimental.pallas import tpu_sc as plsc

C = 16
W = 2048
H = 2048
N = 1048576

NC = 2
NS = 16
NW = NC * NS
L = 16

NQ = N // NW
B = 512
SUB = B // 128
NCHUNK = NQ // B


def _sampler_body(param_hbm, table_hbm, out_hbm,
                  param_v, wx_v, wy_v,
                  i00_v, i10_v, i01_v, i11_v,
                  r00_v, r10_v, r01_v, r11_v,
                  out_v, sem):
  wid = lax.axis_index("s") * NC + lax.axis_index("c")
  iota = lax.iota(jnp.int32, L)

  def chunk_body(ci, _):
    qbase = wid * NQ + ci * B
    pltpu.sync_copy(param_hbm.at[pl.ds(qbase * 2, B * 2)], param_v)

    def prep_body(i, _):
      qoff = i * L
      qv = qoff + iota
      q2 = qv + qv
      u = plsc.load_gather(param_v, [q2])
      v = plsc.load_gather(param_v, [q2 + 1])
      x = u * float(W - 1)
      y = v * float(H - 1)
      x0f = x.astype(jnp.int32)
      y0f = y.astype(jnp.int32)
      wx = x - x0f.astype(jnp.float32)
      wy = y - y0f.astype(jnp.float32)
      x0 = jnp.minimum(jnp.maximum(x0f, 0), W - 1)
      x1 = jnp.minimum(x0 + 1, W - 1)
      y0 = jnp.minimum(jnp.maximum(y0f, 0), H - 1)
      y1 = jnp.minimum(y0 + 1, H - 1)
      xs0 = x0 * H
      xs1 = x1 * H
      wx_v[pl.ds(qoff, L)] = wx
      wy_v[pl.ds(qoff, L)] = wy
      i00_v[pl.ds(qoff, L)] = xs0 + y0
      i10_v[pl.ds(qoff, L)] = xs1 + y0
      i01_v[pl.ds(qoff, L)] = xs0 + y1
      i11_v[pl.ds(qoff, L)] = xs1 + y1
      return 0

    lax.fori_loop(0, B // L, prep_body, 0)

    descs = []
    for ibuf, rbuf in ((i00_v, r00_v), (i10_v, r10_v),
                       (i01_v, r01_v), (i11_v, r11_v)):
      for j in range(SUB):
        descs.append(pltpu.async_copy(
            table_hbm.at[ibuf.at[pl.ds(j * 128, 128)]],
            rbuf.at[pl.ds(j * 128, 128)], sem))
    for d in descs:
      d.wait()

    def comb_body(i, _):
      qoff = i * L
      qv = qoff + iota
      wx = wx_v[pl.ds(qoff, L)]
      wy = wy_v[pl.ds(qoff, L)]
      ob = qv * C
      for c in range(C):
        cc = jnp.full((L,), c, jnp.int32)
        f00 = plsc.load_gather(r00_v, [qv, cc])
        f10 = plsc.load_gather(r10_v, [qv, cc])
        f01 = plsc.load_gather(r01_v, [qv, cc])
        f11 = plsc.load_gather(r11_v, [qv, cc])
        a = f00 + wx * (f10 - f00)
        b = f01 + wx * (f11 - f01)
        o = a + wy * (b - a)
        plsc.store_scatter(out_v, [ob + c], o)
      return 0

    lax.fori_loop(0, B // L, comb_body, 0)

    pltpu.sync_copy(out_v, out_hbm.at[pl.ds(qbase * C, B * C)])
    return 0

  lax.fori_loop(0, NCHUNK, chunk_body, 0)


@jax.jit
def _sampler(param, table):
  mesh = plsc.VectorSubcoreMesh(core_axis_name="c", subcore_axis_name="s",
                                num_cores=NC, num_subcores=NS)
  return pl.kernel(
      _sampler_body,
      out_type=jax.ShapeDtypeStruct((N * C,), jnp.float32),
      mesh=mesh,
      compiler_params=pltpu.CompilerParams(needs_layout_passes=False,
                                           use_tc_tiling_on_sc=False),
      scratch_types=[
          pltpu.VMEM((B * 2,), jnp.float32),
          pltpu.VMEM((B,), jnp.float32),
          pltpu.VMEM((B,), jnp.float32),
          pltpu.VMEM((B,), jnp.int32),
          pltpu.VMEM((B,), jnp.int32),
          pltpu.VMEM((B,), jnp.int32),
          pltpu.VMEM((B,), jnp.int32),
          pltpu.VMEM((B, C), jnp.float32),
          pltpu.VMEM((B, C), jnp.float32),
          pltpu.VMEM((B, C), jnp.float32),
          pltpu.VMEM((B, C), jnp.float32),
          pltpu.VMEM((B * C,), jnp.float32),
          pltpu.SemaphoreType.DMA,
      ],
  )(param, table)


def kernel(input, param):
  table = input.transpose(1, 2, 0).reshape(W * H, C)
  out = _sampler(param.reshape(N * 2), table)
  return out.reshape(N, C)


# speedup vs baseline: 1.0317x; 1.0317x over previous
import jax
import jax.numpy as jnp
from jax import lax
from jax.experimental import pallas as pl
from jax.experimental.pallas import tpu as pltpu
from jax.experimental.pallas import tpu_sc as plsc

C = 16
W = 2048
H = 2048
WH = W * H
N = 1048576

NC = 2
NS = 16
NW = NC * NS
L = 16

NQ = N // NW
B = 256
SUB = B // 128
NCHUNK = NQ // B

XP = W // NS
HB = 1024
NT = XP * (H // HB)


def _body(tex, prm, out, tblf,
          slab, rowb,
          pb0, pb1, wx0, wx1, wy0, wy1,
          ix0, ix1, ix2, ix3, ix4, ix5, ix6, ix7,
          rs0, rs1, rs2, rs3, rs4, rs5, rs6, rs7,
          ob0, ob1,
          sg0, sg1, so0, so1, sp0, sp1, ss0, ss1, sr0, sr1):
  tbl = tblf
  cid = lax.axis_index("c")
  sid = lax.axis_index("s")
  wid = sid * NC + cid
  iota = lax.iota(jnp.int32, L)
  coff = cid * WH

  ss = (ss0, ss1)
  sr = (sr0, sr1)
  sg = (sg0, sg1)
  so = (so0, so1)
  sp = (sp0, sp1)
  pbs = (pb0, pb1)
  wxs = (wx0, wx1)
  wys = (wy0, wy1)
  ixs = ((ix0, ix1, ix2, ix3), (ix4, ix5, ix6, ix7))
  rss = ((rs0, rs1, rs2, rs3), (rs4, rs5, rs6, rs7))
  obs = (ob0, ob1)

  xr0 = sid * XP

  def slab_addr(t):
    x = xr0 + (t >> 1)
    hh = t & 1
    return x, hh

  def fire_slab(t, par):
    x, hh = slab_addr(t)
    for c in range(C):
      pltpu.async_copy(
          tex.at[pl.ds(c, 1), pl.ds(x, 1), pl.ds(hh * HB, HB)],
          slab.at[pl.ds(par * C + c, 1)], ss[par])

  def drain_slab(par):
    for c in range(C):
      pltpu.make_async_copy(tex.at[pl.ds(0, 1), pl.ds(0, 1), pl.ds(0, HB)],
                            slab.at[pl.ds(par * C + c, 1)], ss[par]).wait()

  def fire_rows(t, par):
    x, hh = slab_addr(t)
    pltpu.async_copy(
        rowb.at[pl.ds(par * HB, HB)],
        tbl.at[pl.ds(coff + x * H + hh * HB, HB)], sr[par])

  def drain_rows(par):
    pltpu.make_async_copy(rowb.at[pl.ds(0, HB)],
                          tbl.at[pl.ds(coff, HB)], sr[par]).wait()

  def shuffle(par):
    for c in range(C):
      cv = jnp.full((L,), c, jnp.int32)
      rv0 = par * HB + iota

      def yb8_body(y8, _):
        for u in range(8):
          yb = y8 * 8 + u
          v = slab[par * C + c, 0, pl.ds(yb * L, L)]
          plsc.store_scatter(rowb, [rv0 + yb * L, cv], v)
        return 0

      lax.fori_loop(0, HB // L // 8, yb8_body, 0)

  fire_slab(0, 0)

  def p1_body(t, _):
    for par in (0, 1):
      tt = 2 * t + par

      @pl.when(tt + 1 < NT)
      def _():
        fire_slab(tt + 1, 1 - par)

      drain_slab(par)

      @pl.when(tt >= 2)
      def _():
        drain_rows(par)

      shuffle(par)
      fire_rows(tt, par)
    return 0

  lax.fori_loop(0, NT // 2, p1_body, 0)
  drain_rows(0)
  drain_rows(1)

  plsc.subcore_barrier()

  qw = wid * NQ

  def fire_param(ci, par):
    pltpu.async_copy(prm.at[pl.ds((qw + ci * B) * 2, B * 2)], pbs[par],
                     sp[par])

  def drain_param(par):
    pltpu.make_async_copy(prm.at[pl.ds(0, B * 2)], pbs[par], sp[par]).wait()

  def prep(ci, par):
    def prep_body(i, _):
      qoff = i * L
      qv = qoff + iota
      q2 = qv + qv
      u = plsc.load_gather(pbs[par], [q2])
      v = plsc.load_gather(pbs[par], [q2 + 1])
      x = u * float(W - 1)
      y = v * float(H - 1)
      x0f = x.astype(jnp.int32)
      y0f = y.astype(jnp.int32)
      wx = x - x0f.astype(jnp.float32)
      wy = y - y0f.astype(jnp.float32)
      x0 = jnp.minimum(jnp.maximum(x0f, 0), W - 1)
      x1 = jnp.minimum(x0 + 1, W - 1)
      y0 = jnp.minimum(jnp.maximum(y0f, 0), H - 1)
      y1 = jnp.minimum(y0 + 1, H - 1)
      xs0 = x0 * H + (coff + y0)
      xs1 = x1 * H + (coff + y0)
      dy = y1 - y0
      wxs[par][pl.ds(qoff, L)] = wx
      wys[par][pl.ds(qoff, L)] = wy
      ixs[par][0][pl.ds(qoff, L)] = xs0
      ixs[par][1][pl.ds(qoff, L)] = xs1
      ixs[par][2][pl.ds(qoff, L)] = xs0 + dy
      ixs[par][3][pl.ds(qoff, L)] = xs1 + dy
      return 0

    lax.fori_loop(0, B // L, prep_body, 0)

  def fire_gather(par):
    for k in range(4):
      for j in range(SUB):
        pltpu.async_copy(
            tbl.at[ixs[par][k].at[pl.ds(j * 128, 128)]],
            rss[par][k].at[pl.ds(j * 128, 128)], sg[par])

  def drain_gather(par):
    for k in range(4):
      for j in range(SUB):
        pltpu.make_async_copy(
            tbl.at[ixs[par][k].at[pl.ds(j * 128, 128)]],
            rss[par][k].at[pl.ds(j * 128, 128)], sg[par]).wait()

  def combine(par):
    def comb_body(i, _):
      qoff = i * L
      qv = qoff + iota
      wx = wxs[par][pl.ds(qoff, L)]
      wy = wys[par][pl.ds(qoff, L)]
      for c in range(C):
        cc = jnp.full((L,), c, jnp.int32)
        f00 = plsc.load_gather(rss[par][0], [qv, cc])
        f10 = plsc.load_gather(rss[par][1], [qv, cc])
        f01 = plsc.load_gather(rss[par][2], [qv, cc])
        f11 = plsc.load_gather(rss[par][3], [qv, cc])
        a = f00 + wx * (f10 - f00)
        b = f01 + wx * (f11 - f01)
        o = a + wy * (b - a)
        plsc.store_scatter(obs[par], [qv, cc], o)
      return 0

    lax.fori_loop(0, B // L, comb_body, 0)

  def fire_out(ci, par):
    pltpu.async_copy(obs[par], out.at[pl.ds(qw + ci * B, B)], so[par])

  def drain_out(par):
    pltpu.make_async_copy(obs[par], out.at[pl.ds(0, B)], so[par]).wait()

  fire_param(0, 0)
  drain_param(0)
  prep(0, 0)
  fire_gather(0)
  fire_param(1, 1)

  def p2_body(i, _):
    for par in (0, 1):
      ci = 2 * i + par
      o_ = 1 - par

      @pl.when(ci + 1 < NCHUNK)
      def _():
        drain_param(o_)
        prep(ci + 1, o_)
        fire_gather(o_)

      @pl.when(ci + 2 < NCHUNK)
      def _():
        fire_param(ci + 2, par)

      drain_gather(par)

      @pl.when(ci >= 2)
      def _():
        drain_out(par)

      combine(par)
      fire_out(ci, par)
    return 0

  lax.fori_loop(0, NCHUNK // 2, p2_body, 0)
  drain_out(0)
  drain_out(1)


@jax.jit
def _sampler(tex, prm):
  mesh = plsc.VectorSubcoreMesh(core_axis_name="c", subcore_axis_name="s",
                                num_cores=NC, num_subcores=NS)
  return pl.kernel(
      _body,
      out_type=(jax.ShapeDtypeStruct((N, C), jnp.float32),
                jax.ShapeDtypeStruct((2 * WH, C), jnp.float32)),
      mesh=mesh,
      compiler_params=pltpu.CompilerParams(needs_layout_passes=False,
                                           use_tc_tiling_on_sc=False),
      scratch_types=[
          pltpu.VMEM((2 * C, 1, HB), jnp.float32),
          pltpu.VMEM((2 * HB, C), jnp.float32),
          pltpu.VMEM((B * 2,), jnp.float32),
          pltpu.VMEM((B * 2,), jnp.float32),
          pltpu.VMEM((B,), jnp.float32),
          pltpu.VMEM((B,), jnp.float32),
          pltpu.VMEM((B,), jnp.float32),
          pltpu.VMEM((B,), jnp.float32),
          pltpu.VMEM((B,), jnp.int32),
          pltpu.VMEM((B,), jnp.int32),
          pltpu.VMEM((B,), jnp.int32),
          pltpu.VMEM((B,), jnp.int32),
          pltpu.VMEM((B,), jnp.int32),
          pltpu.VMEM((B,), jnp.int32),
          pltpu.VMEM((B,), jnp.int32),
          pltpu.VMEM((B,), jnp.int32),
          pltpu.VMEM((B, C), jnp.float32),
          pltpu.VMEM((B, C), jnp.float32),
          pltpu.VMEM((B, C), jnp.float32),
          pltpu.VMEM((B, C), jnp.float32),
          pltpu.VMEM((B, C), jnp.float32),
          pltpu.VMEM((B, C), jnp.float32),
          pltpu.VMEM((B, C), jnp.float32),
          pltpu.VMEM((B, C), jnp.float32),
          pltpu.VMEM((B, C), jnp.float32),
          pltpu.VMEM((B, C), jnp.float32),
          pltpu.SemaphoreType.DMA,
          pltpu.SemaphoreType.DMA,
          pltpu.SemaphoreType.DMA,
          pltpu.SemaphoreType.DMA,
          pltpu.SemaphoreType.DMA,
          pltpu.SemaphoreType.DMA,
          pltpu.SemaphoreType.DMA,
          pltpu.SemaphoreType.DMA,
          pltpu.SemaphoreType.DMA,
          pltpu.SemaphoreType.DMA,
      ],
  )(tex, prm)


def kernel(input, param):
  out, _ = _sampler(input, param.reshape(N * 2))
  return out
